# Initial kernel scaffold; baseline (speedup 1.0000x reference)
import functools

import jax
import jax.numpy as jnp
from jax import lax
from jax.experimental import pallas as pl
from jax.experimental.pallas import tpu as pltpu
from jax.experimental.pallas import tpu_sc as plsc

_N = 10000
_E = 320000
_D = 128

_NC = 2
_NS = 16
_NW = _NC * _NS
_EPW = _E // _NW
_CHUNK = 128
_NFULL = _EPW // _CHUNK
_TAIL = _EPW - _NFULL * _CHUNK
_ZROWS = 25
_RPS = _N // _NS


def _sc_aggregate(x, row, col):
    mesh = plsc.VectorSubcoreMesh(core_axis_name="c", subcore_axis_name="s")

    @functools.partial(
        pl.kernel,
        out_type=jax.ShapeDtypeStruct((_NC, _N, _D), jnp.float32),
        mesh=mesh,
        scratch_types=[
            pltpu.VMEM((_CHUNK,), jnp.int32),
            pltpu.VMEM((_CHUNK,), jnp.int32),
            pltpu.VMEM((_CHUNK, _D), jnp.float32),
            pltpu.VMEM((_TAIL,), jnp.int32),
            pltpu.VMEM((_TAIL,), jnp.int32),
            pltpu.VMEM((_TAIL, _D), jnp.float32),
            pltpu.VMEM((_ZROWS, _D), jnp.float32),
            pltpu.VMEM_SHARED((_N, _D), jnp.float32),
        ],
    )
    def agg(x_hbm, row_hbm, col_hbm, out_hbm,
            colv, rowv, gbuf, colt, rowt, gt, zbuf, acc):
        c = lax.axis_index("c")
        s = lax.axis_index("s")
        wid = s * _NC + c

        @pl.loop(0, _ZROWS)
        def _(i):
            @pl.loop(0, _D, step=16)
            def _(j):
                zbuf[i, pl.ds(j, 16)] = jnp.zeros((16,), jnp.float32)

        rbase = s * _RPS

        @pl.loop(0, _RPS, step=_ZROWS)
        def _(r):
            pltpu.sync_copy(zbuf, acc.at[pl.ds(rbase + r, _ZROWS)])

        plsc.subcore_barrier()

        ebase = wid * _EPW

        @pl.loop(0, _NFULL)
        def _(i):
            off = ebase + i * _CHUNK
            pltpu.sync_copy(col_hbm.at[pl.ds(off, _CHUNK)], colv)
            pltpu.sync_copy(row_hbm.at[pl.ds(off, _CHUNK)], rowv)
            pltpu.sync_copy(x_hbm.at[colv], gbuf)
            pltpu.sync_copy(gbuf, acc.at[rowv], add=True)

        offt = ebase + _NFULL * _CHUNK
        pltpu.sync_copy(col_hbm.at[pl.ds(offt, _TAIL)], colt)
        pltpu.sync_copy(row_hbm.at[pl.ds(offt, _TAIL)], rowt)
        pltpu.sync_copy(x_hbm.at[colt], gt)
        pltpu.sync_copy(gt, acc.at[rowt], add=True)

        plsc.subcore_barrier()

        pltpu.sync_copy(acc.at[pl.ds(rbase, _RPS)],
                        out_hbm.at[c, pl.ds(rbase, _RPS)])

    return agg(x, row, col)


_BLK = 1000


def _mm_body(p_ref, w_ref, b_ref, o_ref):
    agg = p_ref[0] + p_ref[1]
    o_ref[...] = jnp.dot(agg, w_ref[...],
                         preferred_element_type=jnp.float32) + b_ref[...]


def _tc_matmul(parts, weight, bias2d):
    return pl.pallas_call(
        _mm_body,
        grid=(_N // _BLK,),
        in_specs=[
            pl.BlockSpec((_NC, _BLK, _D), lambda i: (0, i, 0)),
            pl.BlockSpec((_D, _D), lambda i: (0, 0)),
            pl.BlockSpec((1, _D), lambda i: (0, 0)),
        ],
        out_specs=pl.BlockSpec((_BLK, _D), lambda i: (i, 0)),
        out_shape=jax.ShapeDtypeStruct((_N, _D), jnp.float32),
    )(parts, weight, bias2d)


def kernel(input, edge_index, weight, bias):
    row = edge_index[0].astype(jnp.int32)
    col = edge_index[1].astype(jnp.int32)
    parts = _sc_aggregate(input, row, col)
    return _tc_matmul(parts, weight, bias.reshape(1, _D))


# speedup vs baseline: 6.4352x; 6.4352x over previous
import functools

import jax
import jax.numpy as jnp
from jax import lax
from jax.experimental import pallas as pl
from jax.experimental.pallas import tpu as pltpu
from jax.experimental.pallas import tpu_sc as plsc

_N = 10000
_E = 320000
_D = 128

_NC = 2
_NS = 16
_NW = _NC * _NS
_EPW = _E // _NW
_CHUNK = 128
_NFULL = _EPW // _CHUNK
_TAIL = _EPW - _NFULL * _CHUNK
_ZROWS = 16
_NBLK = _N // _ZROWS


def _sc_aggregate(x, row, col):
    mesh = plsc.VectorSubcoreMesh(core_axis_name="c", subcore_axis_name="s")

    @functools.partial(
        pl.kernel,
        out_type=jax.ShapeDtypeStruct((_NC, _N, _D), jnp.float32),
        mesh=mesh,
        scratch_types=[
            pltpu.VMEM((_CHUNK,), jnp.int32),
            pltpu.VMEM((_CHUNK,), jnp.int32),
            pltpu.VMEM((_CHUNK, _D), jnp.float32),
            pltpu.VMEM((_TAIL,), jnp.int32),
            pltpu.VMEM((_TAIL,), jnp.int32),
            pltpu.VMEM((_TAIL, _D), jnp.float32),
            pltpu.VMEM((_ZROWS, _D), jnp.float32),
            pltpu.VMEM_SHARED((_N, _D), jnp.float32),
        ],
    )
    def agg(x_hbm, row_hbm, col_hbm, out_hbm,
            colv, rowv, gbuf, colt, rowt, gt, zbuf, acc):
        c = lax.axis_index("c")
        s = lax.axis_index("s")
        wid = s * _NC + c

        @pl.loop(0, _ZROWS)
        def _(i):
            @pl.loop(0, _D, step=16)
            def _(j):
                zbuf[i, pl.ds(j, 16)] = jnp.zeros((16,), jnp.float32)

        @pl.loop(s, _NBLK, step=_NS)
        def _(b):
            pltpu.sync_copy(zbuf, acc.at[pl.ds(b * _ZROWS, _ZROWS)])

        plsc.subcore_barrier()

        ebase = wid * _EPW

        @pl.loop(0, _NFULL)
        def _(i):
            off = ebase + i * _CHUNK
            pltpu.sync_copy(col_hbm.at[pl.ds(off, _CHUNK)], colv)
            pltpu.sync_copy(row_hbm.at[pl.ds(off, _CHUNK)], rowv)
            pltpu.sync_copy(x_hbm.at[colv], gbuf)
            pltpu.sync_copy(gbuf, acc.at[rowv], add=True)

        offt = ebase + _NFULL * _CHUNK
        pltpu.sync_copy(col_hbm.at[pl.ds(offt, _TAIL)], colt)
        pltpu.sync_copy(row_hbm.at[pl.ds(offt, _TAIL)], rowt)
        pltpu.sync_copy(x_hbm.at[colt], gt)
        pltpu.sync_copy(gt, acc.at[rowt], add=True)

        plsc.subcore_barrier()

        @pl.loop(s, _NBLK, step=_NS)
        def _(b):
            pltpu.sync_copy(acc.at[pl.ds(b * _ZROWS, _ZROWS)],
                            out_hbm.at[c, pl.ds(b * _ZROWS, _ZROWS)])

    return agg(x, row, col)


_BLK = 1000


def _mm_body(p_ref, w_ref, b_ref, o_ref):
    agg = p_ref[0] + p_ref[1]
    o_ref[...] = jnp.dot(agg, w_ref[...],
                         preferred_element_type=jnp.float32) + b_ref[...]


def _tc_matmul(parts, weight, bias2d):
    return pl.pallas_call(
        _mm_body,
        grid=(_N // _BLK,),
        in_specs=[
            pl.BlockSpec((_NC, _BLK, _D), lambda i: (0, i, 0)),
            pl.BlockSpec((_D, _D), lambda i: (0, 0)),
            pl.BlockSpec((1, _D), lambda i: (0, 0)),
        ],
        out_specs=pl.BlockSpec((_BLK, _D), lambda i: (i, 0)),
        out_shape=jax.ShapeDtypeStruct((_N, _D), jnp.float32),
    )(parts, weight, bias2d)


def kernel(input, edge_index, weight, bias):
    row = edge_index[0].astype(jnp.int32)
    col = edge_index[1].astype(jnp.int32)
    parts = _sc_aggregate(input, row, col)
    return _tc_matmul(parts, weight, bias.reshape(1, _D))
